# baseline (speedup 1.0000x reference)
import functools

import jax
import jax.numpy as jnp
from jax.experimental import pallas as pl
from jax.experimental.pallas import tpu as pltpu

N_BITS = 8
CLIPMIN = 1e-05
QMAX = float(2**N_BITS - 1)

ROWS, COLS = 16384, 2048
BLK_ROWS = 512


def _minmax_body(x_ref, mn_ref, mx_ref):
    i = pl.program_id(0)

    @pl.when(i == 0)
    def _init():
        mn_ref[0, 0] = jnp.inf
        mx_ref[0, 0] = -jnp.inf

    blk = x_ref[...]
    mn_ref[0, 0] = jnp.minimum(mn_ref[0, 0], jnp.min(blk))
    mx_ref[0, 0] = jnp.maximum(mx_ref[0, 0], jnp.max(blk))


def _apply_body(s_ref, x_ref, o_ref):
    scale = s_ref[0, 0]
    zp = s_ref[0, 1]
    x = x_ref[...]
    x_int = jnp.round(x / scale) + zp
    x_quant = jnp.clip(x_int, 0.0, QMAX)
    o_ref[...] = (x_quant - zp) * scale


@jax.jit
def kernel(x):
    rows, cols = x.shape
    grid = (rows // BLK_ROWS,)

    mn, mx = pl.pallas_call(
        _minmax_body,
        grid=grid,
        in_specs=[pl.BlockSpec((BLK_ROWS, cols), lambda i: (i, 0))],
        out_specs=[
            pl.BlockSpec((1, 1), lambda i: (0, 0), memory_space=pltpu.SMEM),
            pl.BlockSpec((1, 1), lambda i: (0, 0), memory_space=pltpu.SMEM),
        ],
        out_shape=[
            jax.ShapeDtypeStruct((1, 1), jnp.float32),
            jax.ShapeDtypeStruct((1, 1), jnp.float32),
        ],
    )(x)

    xmin = jnp.minimum(mn[0, 0], 0.0)
    xmax = jnp.maximum(mx[0, 0], 0.0)
    scale = jnp.clip((xmax - xmin) / QMAX, CLIPMIN, None)
    zero_point = jnp.clip(jnp.round(-xmin / scale), 0.0, QMAX)
    sz = jnp.stack([scale, zero_point]).reshape(1, 2)

    out = pl.pallas_call(
        _apply_body,
        grid=grid,
        in_specs=[
            pl.BlockSpec((1, 2), lambda i: (0, 0), memory_space=pltpu.SMEM),
            pl.BlockSpec((BLK_ROWS, cols), lambda i: (i, 0)),
        ],
        out_specs=pl.BlockSpec((BLK_ROWS, cols), lambda i: (i, 0)),
        out_shape=jax.ShapeDtypeStruct((rows, cols), jnp.float32),
    )(sz, x)
    return out


# speedup vs baseline: 1.0503x; 1.0503x over previous
import jax
import jax.numpy as jnp
from jax.experimental import pallas as pl
from jax.experimental.pallas import tpu as pltpu

N_BITS = 8
CLIPMIN = 1e-05
QMAX = float(2**N_BITS - 1)

RED_BLK = 1024
APP_BLK = 1024


def _minmax_body(x_ref, mn_ref, mx_ref):
    i = pl.program_id(0)

    @pl.when(i == 0)
    def _init():
        mn_ref[0, 0] = jnp.inf
        mx_ref[0, 0] = -jnp.inf

    blk = x_ref[...]
    mn_ref[0, 0] = jnp.minimum(mn_ref[0, 0], jnp.min(blk))
    mx_ref[0, 0] = jnp.maximum(mx_ref[0, 0], jnp.max(blk))


def _apply_body(mn_ref, mx_ref, x_ref, o_ref):
    xmin = jnp.minimum(mn_ref[0, 0], 0.0)
    xmax = jnp.maximum(mx_ref[0, 0], 0.0)
    scale = jnp.maximum((xmax - xmin) / QMAX, CLIPMIN)
    zp = jnp.clip(jnp.round(-xmin / scale), 0.0, QMAX)
    x = x_ref[...]
    x_int = jnp.round(x / scale) + zp
    x_quant = jnp.clip(x_int, 0.0, QMAX)
    o_ref[...] = (x_quant - zp) * scale


@jax.jit
def kernel(x):
    rows, cols = x.shape

    mn, mx = pl.pallas_call(
        _minmax_body,
        grid=(rows // RED_BLK,),
        in_specs=[pl.BlockSpec((RED_BLK, cols), lambda i: (i, 0))],
        out_specs=[
            pl.BlockSpec((1, 1), lambda i: (0, 0), memory_space=pltpu.SMEM),
            pl.BlockSpec((1, 1), lambda i: (0, 0), memory_space=pltpu.SMEM),
        ],
        out_shape=[
            jax.ShapeDtypeStruct((1, 1), jnp.float32),
            jax.ShapeDtypeStruct((1, 1), jnp.float32),
        ],
        compiler_params=pltpu.CompilerParams(
            dimension_semantics=("arbitrary",),
        ),
    )(x)

    out = pl.pallas_call(
        _apply_body,
        grid=(rows // APP_BLK,),
        in_specs=[
            pl.BlockSpec((1, 1), lambda i: (0, 0), memory_space=pltpu.SMEM),
            pl.BlockSpec((1, 1), lambda i: (0, 0), memory_space=pltpu.SMEM),
            pl.BlockSpec((APP_BLK, cols), lambda i: (i, 0)),
        ],
        out_specs=pl.BlockSpec((APP_BLK, cols), lambda i: (i, 0)),
        out_shape=jax.ShapeDtypeStruct((rows, cols), jnp.float32),
        compiler_params=pltpu.CompilerParams(
            dimension_semantics=("parallel",),
        ),
    )(mn, mx, x)
    return out


# speedup vs baseline: 1.1359x; 1.0816x over previous
import jax
import jax.numpy as jnp
from jax.experimental import pallas as pl
from jax.experimental.pallas import tpu as pltpu

N_BITS = 8
CLIPMIN = 1e-05
QMAX = float(2**N_BITS - 1)

RED_BLK = 1024
APP_BLK = 1024


def _minmax_body(x_ref, mn_ref, mx_ref, acc_mn, acc_mx):
    i = pl.program_id(0)
    nblk = pl.num_programs(0)
    blk = x_ref[...]
    b3 = blk.reshape(blk.shape[0] // 8, 8, blk.shape[1])
    pmn = jnp.min(b3, axis=0)
    pmx = jnp.max(b3, axis=0)

    @pl.when(i == 0)
    def _init():
        acc_mn[...] = pmn
        acc_mx[...] = pmx

    @pl.when(i > 0)
    def _acc():
        acc_mn[...] = jnp.minimum(acc_mn[...], pmn)
        acc_mx[...] = jnp.maximum(acc_mx[...], pmx)

    @pl.when(i == nblk - 1)
    def _final():
        mn_ref[0, 0] = jnp.min(acc_mn[...])
        mx_ref[0, 0] = jnp.max(acc_mx[...])


def _apply_body(mn_ref, mx_ref, x_ref, o_ref):
    xmin = jnp.minimum(mn_ref[0, 0], 0.0)
    xmax = jnp.maximum(mx_ref[0, 0], 0.0)
    scale = jnp.maximum((xmax - xmin) / QMAX, CLIPMIN)
    zp = jnp.clip(jnp.round(-xmin / scale), 0.0, QMAX)
    inv = 1.0 / scale
    lo = -zp
    hi = QMAX - zp
    x = x_ref[...]
    q = jnp.clip(jnp.round(x * inv), lo, hi)
    o_ref[...] = q * scale


@jax.jit
def kernel(x):
    rows, cols = x.shape

    mn, mx = pl.pallas_call(
        _minmax_body,
        grid=(rows // RED_BLK,),
        in_specs=[pl.BlockSpec((RED_BLK, cols), lambda i: (i, 0))],
        out_specs=[
            pl.BlockSpec((1, 1), lambda i: (0, 0), memory_space=pltpu.SMEM),
            pl.BlockSpec((1, 1), lambda i: (0, 0), memory_space=pltpu.SMEM),
        ],
        out_shape=[
            jax.ShapeDtypeStruct((1, 1), jnp.float32),
            jax.ShapeDtypeStruct((1, 1), jnp.float32),
        ],
        scratch_shapes=[
            pltpu.VMEM((8, cols), jnp.float32),
            pltpu.VMEM((8, cols), jnp.float32),
        ],
        compiler_params=pltpu.CompilerParams(
            dimension_semantics=("arbitrary",),
        ),
    )(x)

    out = pl.pallas_call(
        _apply_body,
        grid=(rows // APP_BLK,),
        in_specs=[
            pl.BlockSpec((1, 1), lambda i: (0, 0), memory_space=pltpu.SMEM),
            pl.BlockSpec((1, 1), lambda i: (0, 0), memory_space=pltpu.SMEM),
            pl.BlockSpec((APP_BLK, cols), lambda i: (i, 0)),
        ],
        out_specs=pl.BlockSpec((APP_BLK, cols), lambda i: (i, 0)),
        out_shape=jax.ShapeDtypeStruct((rows, cols), jnp.float32),
        compiler_params=pltpu.CompilerParams(
            dimension_semantics=("parallel",),
        ),
    )(mn, mx, x)
    return out


# speedup vs baseline: 1.1663x; 1.0267x over previous
import jax
import jax.numpy as jnp
from jax.experimental import pallas as pl
from jax.experimental.pallas import tpu as pltpu

N_BITS = 8
CLIPMIN = 1e-05
QMAX = float(2**N_BITS - 1)

BLK = 1024


def _fused_body(x_ref, o_ref, acc_mn, acc_mx, s_ref):
    i = pl.program_id(0)
    nb = pl.num_programs(0) // 2

    @pl.when(i < nb)
    def _reduce():
        blk = x_ref[...]
        b3 = blk.reshape(blk.shape[0] // 8, 8, blk.shape[1])
        pmn = jnp.min(b3, axis=0)
        pmx = jnp.max(b3, axis=0)

        @pl.when(i == 0)
        def _init():
            acc_mn[...] = pmn
            acc_mx[...] = pmx

        @pl.when(i > 0)
        def _acc():
            acc_mn[...] = jnp.minimum(acc_mn[...], pmn)
            acc_mx[...] = jnp.maximum(acc_mx[...], pmx)

    @pl.when(i == nb)
    def _scalars():
        xmin = jnp.minimum(jnp.min(acc_mn[...]), 0.0)
        xmax = jnp.maximum(jnp.max(acc_mx[...]), 0.0)
        scale = jnp.maximum((xmax - xmin) / QMAX, CLIPMIN)
        zp = jnp.clip(jnp.round(-xmin / scale), 0.0, QMAX)
        s_ref[0] = scale
        s_ref[1] = zp

    @pl.when(i >= nb)
    def _apply():
        scale = s_ref[0]
        zp = s_ref[1]
        inv = 1.0 / scale
        lo = -zp
        hi = QMAX - zp
        q = jnp.clip(jnp.round(x_ref[...] * inv), lo, hi)
        o_ref[...] = q * scale


@jax.jit
def kernel(x):
    rows, cols = x.shape
    nb = rows // BLK

    out = pl.pallas_call(
        _fused_body,
        grid=(2 * nb,),
        in_specs=[pl.BlockSpec((BLK, cols), lambda i: (i % nb, 0))],
        out_specs=pl.BlockSpec(
            (BLK, cols),
            lambda i: (jnp.where(i < nb, 0, i - nb), 0),
        ),
        out_shape=jax.ShapeDtypeStruct((rows, cols), jnp.float32),
        scratch_shapes=[
            pltpu.VMEM((8, cols), jnp.float32),
            pltpu.VMEM((8, cols), jnp.float32),
            pltpu.SMEM((2,), jnp.float32),
        ],
        compiler_params=pltpu.CompilerParams(
            dimension_semantics=("arbitrary",),
        ),
    )(x)
    return out


# speedup vs baseline: 1.2080x; 1.0358x over previous
import jax
import jax.numpy as jnp
from jax.experimental import pallas as pl
from jax.experimental.pallas import tpu as pltpu

N_BITS = 8
CLIPMIN = 1e-05
QMAX = float(2**N_BITS - 1)

BLK = 512
NC = 8


def _fused_body(x_ref, o_ref, acc_mn, acc_mx, s_ref, cache_ref, sem):
    i = pl.program_id(0)
    nb = pl.num_programs(0) // 2
    nc = cache_ref.shape[0]

    @pl.when(i < nb)
    def _reduce():
        blk = x_ref[...]
        b3 = blk.reshape(blk.shape[0] // 8, 8, blk.shape[1])
        pmn = jnp.min(b3, axis=0)
        pmx = jnp.max(b3, axis=0)

        @pl.when(i == 0)
        def _init():
            acc_mn[...] = pmn
            acc_mx[...] = pmx

        @pl.when(i > 0)
        def _acc():
            acc_mn[...] = jnp.minimum(acc_mn[...], pmn)
            acc_mx[...] = jnp.maximum(acc_mx[...], pmx)

        @pl.when(jnp.logical_and(i >= nb - 1 - nc, i <= nb - 2))
        def _fill_cache():
            slot = i - (nb - 1 - nc)
            cp = pltpu.make_async_copy(x_ref, cache_ref.at[slot], sem)
            cp.start()
            cp.wait()

    @pl.when(i == nb)
    def _scalars():
        xmin = jnp.minimum(jnp.min(acc_mn[...]), 0.0)
        xmax = jnp.maximum(jnp.max(acc_mx[...]), 0.0)
        scale = jnp.maximum((xmax - xmin) / QMAX, CLIPMIN)
        zp = jnp.clip(jnp.round(-xmin / scale), 0.0, QMAX)
        s_ref[0] = scale
        s_ref[1] = zp

    t = 2 * nb - 1 - i

    def _quant(v, scale, zp):
        inv = 1.0 / scale
        return jnp.clip(jnp.round(v * inv), -zp, QMAX - zp) * scale

    @pl.when(jnp.logical_and(i >= nb, jnp.logical_or(t == nb - 1, t < nb - 1 - nc)))
    def _apply_stream():
        o_ref[...] = _quant(x_ref[...], s_ref[0], s_ref[1])

    @pl.when(jnp.logical_and(i >= nb, jnp.logical_and(t >= nb - 1 - nc, t <= nb - 2)))
    def _apply_cached():
        slot = t - (nb - 1 - nc)
        o_ref[...] = _quant(cache_ref[slot], s_ref[0], s_ref[1])


@jax.jit
def kernel(x):
    rows, cols = x.shape
    nb = rows // BLK
    nc = min(NC, nb - 1)

    def in_map(i):
        t = 2 * nb - 1 - i
        src = jnp.where(t >= nb - 1 - nc, nb - 1, t)
        return (jnp.where(i < nb, i, src), 0)

    def out_map(i):
        return (jnp.where(i < nb, nb - 1, 2 * nb - 1 - i), 0)

    out = pl.pallas_call(
        _fused_body,
        grid=(2 * nb,),
        in_specs=[pl.BlockSpec((BLK, cols), in_map)],
        out_specs=pl.BlockSpec((BLK, cols), out_map),
        out_shape=jax.ShapeDtypeStruct((rows, cols), jnp.float32),
        scratch_shapes=[
            pltpu.VMEM((8, cols), jnp.float32),
            pltpu.VMEM((8, cols), jnp.float32),
            pltpu.SMEM((2,), jnp.float32),
            pltpu.VMEM((nc, BLK, cols), jnp.float32),
            pltpu.SemaphoreType.DMA,
        ],
        compiler_params=pltpu.CompilerParams(
            dimension_semantics=("arbitrary",),
        ),
    )(x)
    return out
